# baseline (speedup 1.0000x reference)
import functools
import math

import jax
import jax.numpy as jnp
from jax import lax
from jax.experimental import pallas as pl
from jax.experimental.pallas import tpu as pltpu
from jax.experimental.pallas import tpu_sc as plsc

_D = 64
_SCALE = math.sqrt(_D)
_G = 128
_AB = 2
_NBUF = 2


@functools.lru_cache(maxsize=None)
def _make_kernel(n_i: int, n_j: int, tw: int):
    info = plsc.get_sparse_core_info()
    NC, NS, L = info.num_cores, info.num_subcores, info.num_lanes
    NW = NC * NS
    n_a = n_i // _G
    n_groups = n_a * n_j
    g_per_w = n_groups // NW
    n_stages = g_per_w // _AB
    assert n_groups % NW == 0 and g_per_w % _AB == 0
    assert n_stages % _NBUF == 0 and n_a % _AB == 0 and g_per_w % _AB == 0
    stage_rows = _AB * _G
    tile_elems = _AB * _G * 8
    mesh = plsc.VectorSubcoreMesh(core_axis_name="c", subcore_axis_name="s")

    @functools.partial(
        pl.kernel,
        mesh=mesh,
        compiler_params=pltpu.CompilerParams(
            use_tc_tiling_on_sc=False, needs_layout_passes=False),
        out_type=jax.ShapeDtypeStruct((n_j, _D // 8, n_a // _AB, tile_elems),
                                      jnp.float32),
        scratch_types=[
            pltpu.VMEM((g_per_w, _G), jnp.int32),
            pltpu.VMEM((_NBUF, stage_rows, tw), jnp.float32),
            pltpu.VMEM((_NBUF, _D * stage_rows), jnp.float32),
            pltpu.SemaphoreType.DMA((_NBUF,)),
            pltpu.SemaphoreType.DMA((_NBUF,)),
        ],
    )
    def emb_kernel(idx_hbm, table_hbm, out_hbm, idx_v, rows_v, tr_v,
                   gsem, ssem):
        wid = lax.axis_index("s") * NC + lax.axis_index("c")
        g0 = wid * g_per_w
        pltpu.sync_copy(idx_hbm.at[pl.ds(g0, g_per_w)], idx_v)

        iota = lax.iota(jnp.int32, L)
        cvecs = [iota + c * L for c in range(_D // L)]
        pdst = [[(lax.shift_right_logical(cvecs[c], 3) * (_AB * 1024)
                  + lax.bitwise_and(cvecs[c], 7) * _G + t * (8 * _G))
                 for c in range(_D // L)] for t in range(_AB)]

        def fire_gathers(s, b):
            for t in range(_AB):
                pltpu.async_copy(table_hbm.at[idx_v.at[s * _AB + t]],
                                 rows_v.at[b, pl.ds(t * _G, _G)],
                                 gsem.at[b])

        for b in range(_NBUF):
            fire_gathers(b, b)

        def body(s0):
            for b in range(_NBUF):
                s = s0 + b
                g = g0 + s * _AB
                j = lax.div(g, n_a)
                a = lax.rem(g, n_a)

                pltpu.make_async_copy(table_hbm.at[pl.ds(0, stage_rows)],
                                      rows_v.at[b], gsem.at[b]).wait()

                @pl.when(s >= _NBUF)
                def _():
                    for kb in range(_D // 8):
                        pltpu.make_async_copy(
                            out_hbm.at[0, 0, 0],
                            tr_v.at[b, pl.ds(kb * tile_elems, tile_elems)],
                            ssem.at[b]).wait()

                for t in range(_AB):
                    def tr(r0, _t=t):
                        rvec = lax.bitwise_and(r0 + iota, _G - 1)
                        rfull = rvec + _t * _G
                        for c in range(_D // L):
                            v = plsc.load_gather(rows_v.at[b],
                                                 [rfull, cvecs[c]])
                            plsc.store_scatter(tr_v.at[b],
                                               [pdst[_t][c] + rvec],
                                               v * _SCALE)

                    plsc.parallel_loop(0, _G, unroll=4)(tr)

                a2 = lax.div(a, _AB)
                for kb in range(_D // 8):
                    pltpu.async_copy(
                        tr_v.at[b, pl.ds(kb * tile_elems, tile_elems)],
                        out_hbm.at[j, kb, a2], ssem.at[b])

                @pl.when(s + _NBUF < n_stages)
                def _():
                    fire_gathers(s + _NBUF, b)

        pl.loop(0, n_stages, step=_NBUF)(body)
        for b in range(_NBUF):
            for kb in range(_D // 8):
                pltpu.make_async_copy(
                    out_hbm.at[0, 0, 0],
                    tr_v.at[b, pl.ds(kb * tile_elems, tile_elems)],
                    ssem.at[b]).wait()

    return emb_kernel


def kernel(x, emb_weight):
    n_i, n_j = x.shape
    idx = jnp.reshape(jnp.transpose(x), (n_j * n_i // _G, _G)).astype(jnp.int32)
    padded = jnp.pad(emb_weight, ((0, 0), (0, 2 * _D - emb_weight.shape[1])))
    out4 = _make_kernel(n_i, n_j, 2 * _D)(idx, padded)
    n_a = n_i // _G
    out6 = jnp.reshape(out4, (n_j, _D // 8, n_a // _AB, _AB, 8, _G))
    out = jnp.transpose(out6, (2, 3, 5, 0, 1, 4))
    return jnp.reshape(out, (n_i, n_j, _D))


# speedup vs baseline: 1.0941x; 1.0941x over previous
import functools
import math

import jax
import jax.numpy as jnp
from jax import lax
from jax.experimental import pallas as pl
from jax.experimental.pallas import tpu as pltpu
from jax.experimental.pallas import tpu_sc as plsc

_D = 64
_SCALE = math.sqrt(_D)
_G = 128
_AB = 2
_NBUF = 2


@functools.lru_cache(maxsize=None)
def _make_kernel(n_i: int, n_j: int, tw: int):
    info = plsc.get_sparse_core_info()
    NC, NS, L = info.num_cores, info.num_subcores, info.num_lanes
    NW = NC * NS
    n_a = n_i // _G
    n_groups = n_a * n_j
    g_per_w = n_groups // NW
    n_stages = g_per_w // _AB
    assert n_groups % NW == 0 and g_per_w % _AB == 0
    assert n_stages % _NBUF == 0 and n_a % _AB == 0 and g_per_w % _AB == 0
    stage_rows = _AB * _G
    tile_elems = _AB * _G * 8
    mesh = plsc.VectorSubcoreMesh(core_axis_name="c", subcore_axis_name="s")

    @functools.partial(
        pl.kernel,
        mesh=mesh,
        compiler_params=pltpu.CompilerParams(
            use_tc_tiling_on_sc=False, needs_layout_passes=False),
        out_type=jax.ShapeDtypeStruct((n_j, _D // 8, n_a // _AB, tile_elems),
                                      jnp.float32),
        scratch_types=[
            pltpu.VMEM((g_per_w, _G), jnp.int32),
            pltpu.VMEM((_NBUF, stage_rows, tw), jnp.float32),
            pltpu.VMEM((_NBUF, _D * stage_rows), jnp.float32),
            pltpu.SemaphoreType.DMA((_NBUF,)),
            pltpu.SemaphoreType.DMA((_NBUF,)),
        ],
    )
    def emb_kernel(idx_hbm, table_hbm, out_hbm, idx_v, rows_v, tr_v,
                   gsem, ssem):
        wid = lax.axis_index("s") * NC + lax.axis_index("c")
        g0 = wid * g_per_w
        pltpu.sync_copy(idx_hbm.at[pl.ds(g0, g_per_w)], idx_v)

        iota = lax.iota(jnp.int32, L)
        cvecs = [iota + c * L for c in range(_D // L)]
        pdst = [[(lax.shift_right_logical(cvecs[c], 3) * (_AB * 1024)
                  + lax.bitwise_and(cvecs[c], 7) * _G + t * (8 * _G))
                 for c in range(_D // L)] for t in range(_AB)]

        def fire_gathers(s, b):
            for t in range(_AB):
                pltpu.async_copy(table_hbm.at[idx_v.at[s * _AB + t]],
                                 rows_v.at[b, pl.ds(t * _G, _G)],
                                 gsem.at[b])

        for b in range(_NBUF):
            fire_gathers(b, b)

        def body(s0):
            for b in range(_NBUF):
                s = s0 + b
                g = g0 + s * _AB
                j = lax.div(g, n_a)
                a = lax.rem(g, n_a)

                pltpu.make_async_copy(table_hbm.at[pl.ds(0, stage_rows)],
                                      rows_v.at[b], gsem.at[b]).wait()

                @pl.when(s >= _NBUF)
                def _():
                    for kb in range(_D // 8):
                        pltpu.make_async_copy(
                            out_hbm.at[0, 0, 0],
                            tr_v.at[b, pl.ds(kb * tile_elems, tile_elems)],
                            ssem.at[b]).wait()

                for t in range(_AB):
                    def tr(r0, _t=t):
                        rvec = lax.bitwise_and(r0 + iota, _G - 1)
                        rfull = rvec + _t * _G
                        for c in range(_D // L):
                            v = plsc.load_gather(rows_v.at[b],
                                                 [rfull, cvecs[c]])
                            plsc.store_scatter(tr_v.at[b],
                                               [pdst[_t][c] + rvec],
                                               v * _SCALE)

                    plsc.parallel_loop(0, _G, unroll=4)(tr)

                a2 = lax.div(a, _AB)
                for kb in range(_D // 8):
                    pltpu.async_copy(
                        tr_v.at[b, pl.ds(kb * tile_elems, tile_elems)],
                        out_hbm.at[j, kb, a2], ssem.at[b])

                @pl.when(s + _NBUF < n_stages)
                def _():
                    fire_gathers(s + _NBUF, b)

        pl.loop(0, n_stages, step=_NBUF)(body)
        for b in range(_NBUF):
            for kb in range(_D // 8):
                pltpu.make_async_copy(
                    out_hbm.at[0, 0, 0],
                    tr_v.at[b, pl.ds(kb * tile_elems, tile_elems)],
                    ssem.at[b]).wait()

    return emb_kernel


def kernel(x, emb_weight):
    n_i, n_j = x.shape
    idx = jnp.reshape(jnp.transpose(x * 2),
                      (n_j * n_i // _G, _G)).astype(jnp.int32)
    padded = jnp.pad(emb_weight, ((0, 0), (0, 2 * _D - emb_weight.shape[1])))
    table = jnp.reshape(padded, (2 * padded.shape[0], _D))
    out4 = _make_kernel(n_i, n_j, _D)(idx, table)
    n_a = n_i // _G
    out6 = jnp.reshape(out4, (n_j, _D // 8, n_a // _AB, _AB, 8, _G))
    out = jnp.transpose(out6, (2, 3, 5, 0, 1, 4))
    return jnp.reshape(out, (n_i, n_j, _D))
